# baseline (speedup 1.0000x reference)
import functools

import jax
import jax.numpy as jnp
from jax import lax
from jax.experimental import pallas as pl
from jax.experimental.pallas import tpu as pltpu
from jax.experimental.pallas import tpu_sc as plsc

_NUM_CLASSES = 5
_DIM = 256
_REPEAT = 128
_LANES = 16
_NC = 2
_NS = 16
_NW = _NC * _NS
_ROWS_PER_W = _REPEAT // _NW


def _sc_body(lab_hbm, tab_hbm, out_hbm, lab_v, tab_v, buf_v):
    wid = lax.axis_index("s") * _NC + lax.axis_index("c")
    pltpu.sync_copy(lab_hbm, lab_v.at[pl.ds(0, _NUM_CLASSES)])
    pltpu.sync_copy(tab_hbm, tab_v)
    lv = lab_v[:]
    m = [jnp.where(lv[i] == 1, 1.0, 0.0) for i in range(_NUM_CLASSES)]
    count = m[0] + m[1] + m[2] + m[3] + m[4]
    has_active = count > 0.0
    inv = jnp.where(count > 4.5, 0.2,
          jnp.where(count > 3.5, 0.25,
          jnp.where(count > 2.5, 1.0 / 3.0,
          jnp.where(count > 1.5, 0.5, 1.0))))
    w = [inv * jnp.where(has_active, m[i], 1.0 if i == 0 else 0.0)
         for i in range(_NUM_CLASSES)]

    def chunk_body(c, _):
        sl = pl.ds(c * _LANES, _LANES)
        acc = w[0] * tab_v[0, sl]
        for i in range(1, _NUM_CLASSES):
            acc = acc + w[i] * tab_v[i, sl]

        def row_body(r, _):
            buf_v[r, sl] = acc
            return 0

        return lax.fori_loop(0, _ROWS_PER_W, row_body, 0)

    lax.fori_loop(0, _DIM // _LANES, chunk_body, 0)
    pltpu.sync_copy(buf_v,
                    out_hbm.at[0, pl.ds(wid * _ROWS_PER_W, _ROWS_PER_W)])


@jax.jit
def _run(labels, table):
    f = functools.partial(
        pl.kernel,
        mesh=plsc.VectorSubcoreMesh(core_axis_name="c", subcore_axis_name="s"),
        out_type=jax.ShapeDtypeStruct((1, _REPEAT, _DIM), jnp.float32),
        scratch_types=[
            pltpu.VMEM((_LANES,), jnp.int32),
            pltpu.VMEM((_NUM_CLASSES, _DIM), jnp.float32),
            pltpu.VMEM((_ROWS_PER_W, _DIM), jnp.float32),
        ],
    )(_sc_body)
    return f(labels, table)


def kernel(disease_labels_batch, precomputed_embeddings):
    labels = disease_labels_batch.reshape(-1).astype(jnp.int32)
    return _run(labels, precomputed_embeddings)


# speedup vs baseline: 1.1102x; 1.1102x over previous
import functools

import jax
import jax.numpy as jnp
from jax import lax
from jax.experimental import pallas as pl
from jax.experimental.pallas import tpu as pltpu
from jax.experimental.pallas import tpu_sc as plsc

_NUM_CLASSES = 5
_DIM = 256
_REPEAT = 128
_LANES = 16
_NW = 16
_ROWS_PER_W = _REPEAT // _NW


def _sc_body(lab_hbm, tab_hbm, out_hbm, lab_v, tab_v, buf_v, sem_l, sem_t):
    wid = lax.axis_index("s")
    cp_l = pltpu.make_async_copy(lab_hbm, lab_v.at[pl.ds(0, _NUM_CLASSES)],
                                 sem_l)
    cp_t = pltpu.make_async_copy(tab_hbm, tab_v, sem_t)
    cp_l.start()
    cp_t.start()
    cp_l.wait()
    cp_t.wait()
    lv = lab_v[:]
    m = [jnp.where(lv[i] == 1, 1.0, 0.0) for i in range(_NUM_CLASSES)]
    count = m[0] + m[1] + m[2] + m[3] + m[4]
    has_active = count > 0.0
    inv = jnp.where(count > 4.5, 0.2,
          jnp.where(count > 3.5, 0.25,
          jnp.where(count > 2.5, 1.0 / 3.0,
          jnp.where(count > 1.5, 0.5, 1.0))))
    w = [inv * jnp.where(has_active, m[i], 1.0 if i == 0 else 0.0)
         for i in range(_NUM_CLASSES)]

    def chunk_body(c, _):
        sl = pl.ds(c * _LANES, _LANES)
        acc = w[0] * tab_v[0, sl]
        for i in range(1, _NUM_CLASSES):
            acc = acc + w[i] * tab_v[i, sl]

        def row_body(r, _):
            buf_v[r, sl] = acc
            return 0

        return lax.fori_loop(0, _ROWS_PER_W, row_body, 0)

    lax.fori_loop(0, _DIM // _LANES, chunk_body, 0)
    pltpu.sync_copy(buf_v,
                    out_hbm.at[0, pl.ds(wid * _ROWS_PER_W, _ROWS_PER_W)])


@jax.jit
def _run(labels, table):
    f = functools.partial(
        pl.kernel,
        mesh=plsc.VectorSubcoreMesh(core_axis_name="c", subcore_axis_name="s",
                                    num_cores=1),
        out_type=jax.ShapeDtypeStruct((1, _REPEAT, _DIM), jnp.float32),
        scratch_types=[
            pltpu.VMEM((_LANES,), jnp.int32),
            pltpu.VMEM((_NUM_CLASSES, _DIM), jnp.float32),
            pltpu.VMEM((_ROWS_PER_W, _DIM), jnp.float32),
            pltpu.SemaphoreType.DMA,
            pltpu.SemaphoreType.DMA,
        ],
    )(_sc_body)
    return f(labels, table)


def kernel(disease_labels_batch, precomputed_embeddings):
    labels = disease_labels_batch.reshape(-1).astype(jnp.int32)
    return _run(labels, precomputed_embeddings)
